# baseline (speedup 1.0000x reference)
import jax
import jax.numpy as jnp
from jax.experimental import pallas as pl
from jax.experimental.pallas import tpu as pltpu

BATCH = 16384
POS = 10
NEG = 50
JPAD = 64
TC_BLK = 2048


def _loss_body(dots_ref, out_ref):
    d = dots_ref[...]
    row = jax.lax.broadcasted_iota(jnp.int32, d.shape, 0)
    sign = jnp.where(row < POS, 1.0, -1.0)
    x = d * sign
    ls = jnp.minimum(x, 0.0) - jnp.log1p(jnp.exp(-jnp.abs(x)))
    contrib = jnp.where(row < POS + NEG, ls, 0.0)
    out_ref[...] = -jnp.sum(contrib, axis=0, keepdims=True)


def _loss_from_dots(dots):
    out = pl.pallas_call(
        _loss_body,
        grid=(BATCH // TC_BLK,),
        in_specs=[pl.BlockSpec((JPAD, TC_BLK), lambda i: (0, i))],
        out_specs=pl.BlockSpec((1, TC_BLK), lambda i: (0, i)),
        out_shape=jax.ShapeDtypeStruct((1, BATCH), jnp.float32),
    )(dots)
    return out.reshape(BATCH)


def kernel(input_labels, pos_labels, neg_labels, in_embed, out_embed):
    inp = jnp.take(in_embed, input_labels, axis=0)
    pos = jnp.take(out_embed, pos_labels, axis=0)
    neg = jnp.take(out_embed, neg_labels, axis=0)
    dp = jnp.einsum("bpd,bd->bp", pos, inp)
    dn = jnp.einsum("bnd,bd->bn", neg, inp)
    dots = jnp.concatenate(
        [dp, dn, jnp.zeros((BATCH, JPAD - POS - NEG), jnp.float32)], axis=1)
    return _loss_from_dots(dots.T)


# speedup vs baseline: 2.1609x; 2.1609x over previous
import functools

import jax
import jax.numpy as jnp
from jax import lax
from jax.experimental import pallas as pl
from jax.experimental.pallas import tpu as pltpu
from jax.experimental.pallas import tpu_sc as plsc

BATCH = 16384
EMBED = 64
POS = 10
NEG = 50
J = POS + NEG
JPAD = 64
TC_BLK = 2048

_info = plsc.get_sparse_core_info()
_NC, _NS, _L = _info.num_cores, _info.num_subcores, _info.num_lanes
NW = _NC * _NS
BPW = BATCH // NW
C = 16
NCHUNK = BPW // C
STR_ROWS = 96
NSTR = (C * J) // STR_ROWS


def _sc_dots(in_labels, out_labels, in_embed, out_embed):
    mesh = plsc.VectorSubcoreMesh(core_axis_name="c", subcore_axis_name="s")

    @functools.partial(
        pl.kernel,
        out_type=jax.ShapeDtypeStruct((JPAD, BATCH), jnp.float32),
        mesh=mesh,
        compiler_params=pltpu.CompilerParams(
            needs_layout_passes=False, use_tc_tiling_on_sc=False),
        scratch_types=[
            pltpu.VMEM((C,), jnp.int32),
            pltpu.VMEM((C * J,), jnp.int32),
            pltpu.VMEM((C, EMBED), jnp.float32),
            pltpu.VMEM((C * J, EMBED), jnp.float32),
            pltpu.VMEM((EMBED, C), jnp.float32),
            pltpu.VMEM((JPAD, 8 * C), jnp.float32),
            pltpu.SemaphoreType.DMA,
        ],
    )
    def k(in_lab_hbm, out_lab_hbm, in_emb_hbm, out_emb_hbm, dots_hbm,
          idx_in_v, idx_out_v, rows_in_v, rows_out_v, ivT_v, dots_v, sem):
        wid = lax.axis_index("s") * _NC + lax.axis_index("c")
        base = wid * BPW
        zero = jnp.zeros((_L,), jnp.float32)
        iota = lax.iota(jnp.int32, _L)
        for jj in range(J, JPAD):
            for q in range(8):
                dots_v[jj, pl.ds(q * _L, _L)] = zero

        def chunk(fg, gc):
            g = fg * 8 + gc
            b0 = base + g * C
            pltpu.sync_copy(in_lab_hbm.at[pl.ds(b0, C)], idx_in_v)
            pltpu.sync_copy(out_lab_hbm.at[pl.ds(b0 * J, C * J)], idx_out_v)
            cps = [pltpu.async_copy(in_emb_hbm.at[idx_in_v], rows_in_v, sem)]
            for s in range(NSTR):
                cps.append(pltpu.async_copy(
                    out_emb_hbm.at[idx_out_v.at[pl.ds(s * STR_ROWS, STR_ROWS)]],
                    rows_out_v.at[pl.ds(s * STR_ROWS, STR_ROWS)], sem))
            for cp in cps:
                cp.wait()

            def tbody(d, carry_t):
                dvec = jnp.broadcast_to(d, (_L,)).astype(jnp.int32)
                ivT_v[d, :] = plsc.load_gather(rows_in_v, [iota, dvec])
                return carry_t

            lax.fori_loop(0, EMBED, tbody, 0)

            for jb in range(4):
                rvecs = [iota * J + (jb * 15 + jj) for jj in range(15)]

                def dbody(d, accs, rvecs=rvecs):
                    dvec = jnp.broadcast_to(d, (_L,)).astype(jnp.int32)
                    iv = ivT_v[d, :]
                    return tuple(
                        accs[jj]
                        + plsc.load_gather(rows_out_v, [rvecs[jj], dvec]) * iv
                        for jj in range(15))

                accs = lax.fori_loop(0, EMBED, dbody,
                                     tuple(zero for _ in range(15)))
                for jj in range(15):
                    dots_v[jb * 15 + jj, pl.ds(gc * C, C)] = accs[jj]

        def flush_group(fg, carry):
            def inner(gc, carry_i):
                chunk(fg, gc)
                return carry_i

            lax.fori_loop(0, 8, inner, 0)
            pltpu.sync_copy(dots_v,
                            dots_hbm.at[:, pl.ds(base + fg * 8 * C, 8 * C)])
            return carry

        lax.fori_loop(0, NCHUNK // 8, flush_group, 0)

    return k(in_labels, out_labels, in_embed, out_embed)


def _loss_body(dots_ref, out_ref):
    d = dots_ref[...]
    row = jax.lax.broadcasted_iota(jnp.int32, d.shape, 0)
    sign = jnp.where(row < POS, 1.0, -1.0)
    x = d * sign
    ls = jnp.minimum(x, 0.0) - jnp.log1p(jnp.exp(-jnp.abs(x)))
    contrib = jnp.where(row < POS + NEG, ls, 0.0)
    out_ref[...] = -jnp.sum(contrib, axis=0, keepdims=True)


def _loss_from_dots(dots):
    out = pl.pallas_call(
        _loss_body,
        grid=(BATCH // TC_BLK,),
        in_specs=[pl.BlockSpec((JPAD, TC_BLK), lambda i: (0, i))],
        out_specs=pl.BlockSpec((1, TC_BLK), lambda i: (0, i)),
        out_shape=jax.ShapeDtypeStruct((1, BATCH), jnp.float32),
    )(dots)
    return out.reshape(BATCH)


def kernel(input_labels, pos_labels, neg_labels, in_embed, out_embed):
    out_labels = jnp.concatenate([pos_labels, neg_labels], axis=1).reshape(-1)
    dots = _sc_dots(input_labels.astype(jnp.int32),
                    out_labels.astype(jnp.int32), in_embed, out_embed)
    return _loss_from_dots(dots)
